# baseline (speedup 1.0000x reference)
import functools

import jax
import jax.numpy as jnp
from jax import lax
from jax.experimental import pallas as pl
from jax.experimental.pallas import tpu as pltpu
from jax.experimental.pallas import tpu_sc as plsc

_B = 16384
_H = 200
_V = 1000000
_D = 64
_C = 2

_NC = 2
_NS = 16
_NW = _NC * _NS

_IDX_PER_STREAM = 128
_NSTREAM = (_B * _H) // _IDX_PER_STREAM
_STREAMS_PER_TILE = _NSTREAM // _NW
_K = 16
_CHUNKS = _STREAMS_PER_TILE // _K

_QROWS = 7816
_VPAD = _QROWS * 128
_STAGE = _VPAD // _NS


def _project_body(emb_ref, w_ref, out_ref):
    e = emb_ref[...]
    w = w_ref[...]
    dw = w[0:1, :] - w[1:2, :]
    q = lax.dot_general(e, dw, (((1,), (1,)), ((), ())),
                        preferred_element_type=jnp.float32)
    out_ref[...] = q.reshape(8, 128) * (1.0 / _H)


def _project(emb, w):
    rows_per_blk = 8 * 128
    grid = _QROWS // 8
    return pl.pallas_call(
        _project_body,
        grid=(grid,),
        in_specs=[
            pl.BlockSpec((rows_per_blk, _D), lambda i: (i, 0)),
            pl.BlockSpec((_C, _D), lambda i: (0, 0)),
        ],
        out_specs=pl.BlockSpec((8, 128), lambda i: (i, 0)),
        out_shape=jax.ShapeDtypeStruct((_QROWS, 128), jnp.float32),
    )(emb, w)


def _gather_body(xr_hbm, q_hbm, out_hbm,
                 q_sh, idx_a, idx_b, rows_a, rows_b,
                 stage_sem, isem_a, isem_b, gsem_a, gsem_b, ssem_a, ssem_b):
    sid = lax.axis_index("s")
    wid = sid * _NC + lax.axis_index("c")
    base = wid * _STREAMS_PER_TILE

    st0 = sid * _STAGE
    pltpu.async_copy(q_hbm.at[pl.ds(st0, _STAGE)],
                     q_sh.at[pl.ds(st0, _STAGE)], stage_sem).wait()
    plsc.subcore_barrier()

    def idx_slice(t):
        return xr_hbm.at[pl.ds(base + t * _K, _K)]

    def out_slice(t):
        return out_hbm.at[pl.ds(base + t * _K, _K)]

    pltpu.async_copy(idx_slice(0), idx_a, isem_a)
    pltpu.async_copy(idx_slice(1), idx_b, isem_b)

    def half_step(t, last_t, idx_v, rows_v, isem, gsem, ssem):
        pltpu.make_async_copy(idx_slice(t), idx_v, isem).wait()

        @pl.when(t >= 2)
        def _():
            pltpu.make_async_copy(rows_v, out_slice(t - 2), ssem).wait()

        cps = [
            pltpu.async_copy(q_sh.at[idx_v.at[k]], rows_v.at[k], gsem)
            for k in range(_K)
        ]
        for cp in cps:
            cp.wait()

        @pl.when(t + 2 <= last_t)
        def _():
            pltpu.async_copy(idx_slice(t + 2), idx_v, isem)

        pltpu.async_copy(rows_v, out_slice(t), ssem)

    def dstep(m, carry):
        half_step(2 * m, _CHUNKS - 1, idx_a, rows_a, isem_a, gsem_a, ssem_a)
        half_step(2 * m + 1, _CHUNKS - 1, idx_b, rows_b, isem_b, gsem_b, ssem_b)
        return carry

    lax.fori_loop(0, _CHUNKS // 2, dstep, 0)

    pltpu.make_async_copy(rows_a, out_slice(_CHUNKS - 2), ssem_a).wait()
    pltpu.make_async_copy(rows_b, out_slice(_CHUNKS - 1), ssem_b).wait()


def _gather(xr, qp):
    mesh = plsc.VectorSubcoreMesh(core_axis_name="c", subcore_axis_name="s")
    kern = functools.partial(
        pl.kernel,
        mesh=mesh,
        out_type=jax.ShapeDtypeStruct((_NSTREAM, _IDX_PER_STREAM),
                                      jnp.float32),
        scratch_types=[
            pltpu.VMEM_SHARED((_VPAD,), jnp.float32),
            pltpu.VMEM((_K, _IDX_PER_STREAM), jnp.int32),
            pltpu.VMEM((_K, _IDX_PER_STREAM), jnp.int32),
            pltpu.VMEM((_K, _IDX_PER_STREAM), jnp.float32),
            pltpu.VMEM((_K, _IDX_PER_STREAM), jnp.float32),
            pltpu.SemaphoreType.DMA,
            pltpu.SemaphoreType.DMA,
            pltpu.SemaphoreType.DMA,
            pltpu.SemaphoreType.DMA,
            pltpu.SemaphoreType.DMA,
            pltpu.SemaphoreType.DMA,
            pltpu.SemaphoreType.DMA,
        ],
        compiler_params=pltpu.CompilerParams(use_tc_tiling_on_sc=False),
    )(_gather_body)
    return kern(xr, qp)


def _head_body(g_ref, b_ref, out_ref):
    g = g_ref[...]
    bb = b_ref[...]
    d = jnp.sum(g, axis=1, keepdims=True) + (bb[0, 0] - bb[0, 1])
    c2 = lax.broadcasted_iota(jnp.int32, (g.shape[0], _C), 1)
    z = jnp.where(c2 == 0, -d, d)
    sp = jnp.maximum(z, 0.0) + jnp.log1p(jnp.exp(-jnp.abs(z)))
    out_ref[...] = -sp


def _head(g, b2):
    blk = 2048
    grid = _B // blk
    return pl.pallas_call(
        _head_body,
        grid=(grid,),
        in_specs=[
            pl.BlockSpec((blk, _H), lambda i: (i, 0)),
            pl.BlockSpec((1, _C), lambda i: (0, 0)),
        ],
        out_specs=pl.BlockSpec((blk, _C), lambda i: (i, 0)),
        out_shape=jax.ShapeDtypeStruct((_B, _C), jnp.float32),
    )(g, b2)


def kernel(x, emb_table, W, b):
    qp = _project(emb_table, W).reshape(_VPAD)
    xr = x.reshape(_NSTREAM, _IDX_PER_STREAM)
    g = _gather(xr, qp)
    return _head(g.reshape(_B, _H), b.reshape(1, _C))


# speedup vs baseline: 4.6809x; 4.6809x over previous
import functools

import jax
import jax.numpy as jnp
from jax import lax
from jax.experimental import pallas as pl
from jax.experimental.pallas import tpu as pltpu
from jax.experimental.pallas import tpu_sc as plsc

_B = 16384
_H = 200
_V = 1000000
_D = 64
_C = 2

_NC = 2
_NS = 16
_NW = _NC * _NS

_IDX_PER_STREAM = 128
_NSTREAM = (_B * _H) // _IDX_PER_STREAM
_STREAMS_PER_TILE = _NSTREAM // _NW
_K = 16
_CHUNKS = _STREAMS_PER_TILE // _K

_STAGE = 62504
_LAST = _V - 15 * _STAGE


def _project_body(embt_ref, wt_ref, out_ref):
    e = embt_ref[...]
    wt = wt_ref[...]
    dwv = (wt[:, 0:1] - wt[:, 1:2]) * (1.0 / _H)
    out_ref[...] = lax.dot_general(dwv, e, (((0,), (0,)), ((), ())),
                                   preferred_element_type=jnp.float32)


def _project(embt, wt):
    blk = 12800
    grid = -(-_V // blk)
    return pl.pallas_call(
        _project_body,
        grid=(grid,),
        in_specs=[
            pl.BlockSpec((_D, blk), lambda i: (0, i)),
            pl.BlockSpec((_D, _C), lambda i: (0, 0)),
        ],
        out_specs=pl.BlockSpec((1, blk), lambda i: (0, i)),
        out_shape=jax.ShapeDtypeStruct((1, _V), jnp.float32),
    )(embt, wt)


def _gather_body(xr_hbm, q_hbm, out_hbm,
                 q_sh, idx_a, idx_b, rows_a, rows_b,
                 stage_sem, isem_a, isem_b, gsem_a, gsem_b, ssem_a, ssem_b):
    sid = lax.axis_index("s")
    wid = sid * _NC + lax.axis_index("c")
    base = wid * _STREAMS_PER_TILE

    @pl.when(sid < _NS - 1)
    def _():
        st0 = sid * _STAGE
        pltpu.async_copy(q_hbm.at[pl.ds(st0, _STAGE)],
                         q_sh.at[pl.ds(st0, _STAGE)], stage_sem).wait()

    @pl.when(sid == _NS - 1)
    def _():
        st0 = (_NS - 1) * _STAGE
        pltpu.async_copy(q_hbm.at[pl.ds(st0, _LAST)],
                         q_sh.at[pl.ds(st0, _LAST)], stage_sem).wait()

    plsc.subcore_barrier()

    def idx_slice(t):
        return xr_hbm.at[pl.ds(base + t * _K, _K)]

    def out_slice(t):
        return out_hbm.at[pl.ds(base + t * _K, _K)]

    pltpu.async_copy(idx_slice(0), idx_a, isem_a)
    pltpu.async_copy(idx_slice(1), idx_b, isem_b)

    def half_step(t, last_t, idx_v, rows_v, isem, gsem, ssem):
        pltpu.make_async_copy(idx_slice(t), idx_v, isem).wait()

        @pl.when(t >= 2)
        def _():
            pltpu.make_async_copy(rows_v, out_slice(t - 2), ssem).wait()

        cps = [
            pltpu.async_copy(q_sh.at[idx_v.at[k]], rows_v.at[k], gsem)
            for k in range(_K)
        ]
        for cp in cps:
            cp.wait()

        @pl.when(t + 2 <= last_t)
        def _():
            pltpu.async_copy(idx_slice(t + 2), idx_v, isem)

        pltpu.async_copy(rows_v, out_slice(t), ssem)

    def dstep(m, carry):
        half_step(2 * m, _CHUNKS - 1, idx_a, rows_a, isem_a, gsem_a, ssem_a)
        half_step(2 * m + 1, _CHUNKS - 1, idx_b, rows_b, isem_b, gsem_b, ssem_b)
        return carry

    lax.fori_loop(0, _CHUNKS // 2, dstep, 0)

    pltpu.make_async_copy(rows_a, out_slice(_CHUNKS - 2), ssem_a).wait()
    pltpu.make_async_copy(rows_b, out_slice(_CHUNKS - 1), ssem_b).wait()


def _gather(xr, qp):
    mesh = plsc.VectorSubcoreMesh(core_axis_name="c", subcore_axis_name="s")
    kern = functools.partial(
        pl.kernel,
        mesh=mesh,
        out_type=jax.ShapeDtypeStruct((_NSTREAM, _IDX_PER_STREAM),
                                      jnp.float32),
        scratch_types=[
            pltpu.VMEM_SHARED((_V,), jnp.float32),
            pltpu.VMEM((_K, _IDX_PER_STREAM), jnp.int32),
            pltpu.VMEM((_K, _IDX_PER_STREAM), jnp.int32),
            pltpu.VMEM((_K, _IDX_PER_STREAM), jnp.float32),
            pltpu.VMEM((_K, _IDX_PER_STREAM), jnp.float32),
            pltpu.SemaphoreType.DMA,
            pltpu.SemaphoreType.DMA,
            pltpu.SemaphoreType.DMA,
            pltpu.SemaphoreType.DMA,
            pltpu.SemaphoreType.DMA,
            pltpu.SemaphoreType.DMA,
            pltpu.SemaphoreType.DMA,
        ],
        compiler_params=pltpu.CompilerParams(use_tc_tiling_on_sc=False),
    )(_gather_body)
    return kern(xr, qp)


def _head_body(gt_ref, b_ref, out_ref):
    g = gt_ref[...]
    bb = b_ref[...]
    d = jnp.sum(g, axis=0, keepdims=True) + (bb[0, 0] - bb[0, 1])
    r2 = lax.broadcasted_iota(jnp.int32, (_C, g.shape[1]), 0)
    z = jnp.where(r2 == 0, -d, d)
    sp = jnp.maximum(z, 0.0) + jnp.log1p(jnp.exp(-jnp.abs(z)))
    out_ref[...] = -sp


def _head(gt, b2):
    blk = 2048
    grid = _B // blk
    return pl.pallas_call(
        _head_body,
        grid=(grid,),
        in_specs=[
            pl.BlockSpec((_H, blk), lambda i: (0, i)),
            pl.BlockSpec((1, _C), lambda i: (0, 0)),
        ],
        out_specs=pl.BlockSpec((_C, blk), lambda i: (0, i)),
        out_shape=jax.ShapeDtypeStruct((_C, _B), jnp.float32),
    )(gt, b2)


def kernel(x, emb_table, W, b):
    q = _project(emb_table.T, W.T).reshape(_V)
    xr = x.T.reshape(_NSTREAM, _IDX_PER_STREAM)
    g = _gather(xr, q)
    out_t = _head(g.reshape(_H, _B), b.reshape(1, _C))
    return out_t.T


# speedup vs baseline: 5.1703x; 1.1045x over previous
import functools

import jax
import jax.numpy as jnp
from jax import lax
from jax.experimental import pallas as pl
from jax.experimental.pallas import tpu as pltpu
from jax.experimental.pallas import tpu_sc as plsc

_B = 16384
_H = 200
_V = 1000000
_D = 64
_C = 2

_NC = 2
_NS = 16
_NW = _NC * _NS

_IDX_PER_STREAM = 128
_NSTREAM = (_B * _H) // _IDX_PER_STREAM
_STREAMS_PER_TILE = _NSTREAM // _NW
_K = 20
_CHUNKS = _STREAMS_PER_TILE // _K

_STAGE = 62504
_LAST = _V - 15 * _STAGE


def _project_body(embt_ref, wt_ref, out_ref):
    e = embt_ref[...]
    wt = wt_ref[...]
    dwv = (wt[:, 0:1] - wt[:, 1:2]) * (1.0 / _H)
    out_ref[...] = lax.dot_general(dwv, e, (((0,), (0,)), ((), ())),
                                   preferred_element_type=jnp.float32)


def _project(embt, wt):
    blk = 25600
    grid = -(-_V // blk)
    return pl.pallas_call(
        _project_body,
        grid=(grid,),
        in_specs=[
            pl.BlockSpec((_D, blk), lambda i: (0, i)),
            pl.BlockSpec((_D, _C), lambda i: (0, 0)),
        ],
        out_specs=pl.BlockSpec((1, blk), lambda i: (0, i)),
        out_shape=jax.ShapeDtypeStruct((1, _V), jnp.float32),
    )(embt, wt)


def _gather_body(xr_hbm, q_hbm, out_hbm,
                 q_sh, idx_a, idx_b, rows_a, rows_b,
                 stage_sem, isem_a, isem_b, gsem_a, gsem_b, ssem_a, ssem_b):
    sid = lax.axis_index("s")
    wid = sid * _NC + lax.axis_index("c")
    base = wid * _STREAMS_PER_TILE

    @pl.when(sid < _NS - 1)
    def _():
        st0 = sid * _STAGE
        pltpu.async_copy(q_hbm.at[pl.ds(st0, _STAGE)],
                         q_sh.at[pl.ds(st0, _STAGE)], stage_sem).wait()

    @pl.when(sid == _NS - 1)
    def _():
        st0 = (_NS - 1) * _STAGE
        pltpu.async_copy(q_hbm.at[pl.ds(st0, _LAST)],
                         q_sh.at[pl.ds(st0, _LAST)], stage_sem).wait()

    plsc.subcore_barrier()

    def idx_slice(t):
        return xr_hbm.at[pl.ds(base + t * _K, _K)]

    def out_slice(t):
        return out_hbm.at[pl.ds(base + t * _K, _K)]

    pltpu.async_copy(idx_slice(0), idx_a, isem_a)
    pltpu.async_copy(idx_slice(1), idx_b, isem_b)

    def half_step(t, last_t, idx_v, rows_v, isem, gsem, ssem):
        pltpu.make_async_copy(idx_slice(t), idx_v, isem).wait()

        @pl.when(t >= 2)
        def _():
            pltpu.make_async_copy(rows_v, out_slice(t - 2), ssem).wait()

        cps = [
            pltpu.async_copy(q_sh.at[idx_v.at[k]], rows_v.at[k], gsem)
            for k in range(_K)
        ]
        for cp in cps:
            cp.wait()

        @pl.when(t + 2 <= last_t)
        def _():
            pltpu.async_copy(idx_slice(t + 2), idx_v, isem)

        pltpu.async_copy(rows_v, out_slice(t), ssem)

    def dstep(m, carry):
        half_step(2 * m, _CHUNKS - 1, idx_a, rows_a, isem_a, gsem_a, ssem_a)
        half_step(2 * m + 1, _CHUNKS - 1, idx_b, rows_b, isem_b, gsem_b, ssem_b)
        return carry

    lax.fori_loop(0, _CHUNKS // 2, dstep, 0)

    pltpu.make_async_copy(rows_a, out_slice(_CHUNKS - 2), ssem_a).wait()
    pltpu.make_async_copy(rows_b, out_slice(_CHUNKS - 1), ssem_b).wait()


def _gather(xr, qp):
    mesh = plsc.VectorSubcoreMesh(core_axis_name="c", subcore_axis_name="s")
    kern = functools.partial(
        pl.kernel,
        mesh=mesh,
        out_type=jax.ShapeDtypeStruct((_NSTREAM, _IDX_PER_STREAM),
                                      jnp.float32),
        scratch_types=[
            pltpu.VMEM_SHARED((_V,), jnp.float32),
            pltpu.VMEM((_K, _IDX_PER_STREAM), jnp.int32),
            pltpu.VMEM((_K, _IDX_PER_STREAM), jnp.int32),
            pltpu.VMEM((_K, _IDX_PER_STREAM), jnp.float32),
            pltpu.VMEM((_K, _IDX_PER_STREAM), jnp.float32),
            pltpu.SemaphoreType.DMA,
            pltpu.SemaphoreType.DMA,
            pltpu.SemaphoreType.DMA,
            pltpu.SemaphoreType.DMA,
            pltpu.SemaphoreType.DMA,
            pltpu.SemaphoreType.DMA,
            pltpu.SemaphoreType.DMA,
        ],
        compiler_params=pltpu.CompilerParams(use_tc_tiling_on_sc=False),
    )(_gather_body)
    return kern(xr, qp)


def _head_body(gt_ref, b_ref, out_ref):
    g = gt_ref[...]
    bb = b_ref[...]
    d = jnp.sum(g, axis=0, keepdims=True) + (bb[0, 0] - bb[0, 1])
    r2 = lax.broadcasted_iota(jnp.int32, (_C, g.shape[1]), 0)
    z = jnp.where(r2 == 0, -d, d)
    sp = jnp.maximum(z, 0.0) + jnp.log1p(jnp.exp(-jnp.abs(z)))
    out_ref[...] = -sp


def _head(gt, b2):
    blk = 2048
    grid = _B // blk
    return pl.pallas_call(
        _head_body,
        grid=(grid,),
        in_specs=[
            pl.BlockSpec((_H, blk), lambda i: (0, i)),
            pl.BlockSpec((1, _C), lambda i: (0, 0)),
        ],
        out_specs=pl.BlockSpec((_C, blk), lambda i: (0, i)),
        out_shape=jax.ShapeDtypeStruct((_C, _B), jnp.float32),
    )(gt, b2)


def kernel(x, emb_table, W, b):
    q = _project(emb_table.T, W.T).reshape(_V)
    xr = x.T.reshape(_NSTREAM, _IDX_PER_STREAM)
    g = _gather(xr, q)
    out_t = _head(g.reshape(_H, _B), b.reshape(1, _C))
    return out_t.T


# speedup vs baseline: 7.0755x; 1.3685x over previous
import functools

import jax
import jax.numpy as jnp
from jax import lax
from jax.experimental import pallas as pl
from jax.experimental.pallas import tpu as pltpu
from jax.experimental.pallas import tpu_sc as plsc

_B = 16384
_H = 200
_V = 1000000
_D = 64
_C = 2

_NC = 2
_NS = 16
_NW = _NC * _NS

_IDX_PER_STREAM = 128
_NSTREAM = (_B * _H) // _IDX_PER_STREAM
_STREAMS_PER_TILE = _NSTREAM // _NW
_K = 20
_CHUNKS = _STREAMS_PER_TILE // _K

_STAGE = 62504
_LAST = _V - 15 * _STAGE


def _project_body(embt_ref, wt_ref, out_ref):
    e = embt_ref[...]
    wt = wt_ref[...]
    dwv = (wt[:, 0:1] - wt[:, 1:2]) * (1.0 / _H)
    p = lax.dot_general(dwv, e, (((0,), (0,)), ((), ())),
                        preferred_element_type=jnp.float32)
    out_ref[...] = p.reshape(p.shape[1])


def _project(embt, wt):
    blk = 25600
    grid = -(-_V // blk)
    return pl.pallas_call(
        _project_body,
        grid=(grid,),
        in_specs=[
            pl.BlockSpec((_D, blk), lambda i: (0, i)),
            pl.BlockSpec((_D, _C), lambda i: (0, 0)),
        ],
        out_specs=pl.BlockSpec((blk,), lambda i: (i,)),
        out_shape=jax.ShapeDtypeStruct((_V,), jnp.float32),
    )(embt, wt)


def _gather_body(xr_hbm, q_hbm, out_hbm,
                 q_sh, idx_a, idx_b, rows_a, rows_b,
                 stage_sem, isem_a, isem_b, gsem_a, gsem_b, ssem_a, ssem_b):
    sid = lax.axis_index("s")
    wid = sid * _NC + lax.axis_index("c")
    base = wid * _STREAMS_PER_TILE

    @pl.when(sid < _NS - 1)
    def _():
        st0 = sid * _STAGE
        pltpu.async_copy(q_hbm.at[pl.ds(st0, _STAGE)],
                         q_sh.at[pl.ds(st0, _STAGE)], stage_sem).wait()

    @pl.when(sid == _NS - 1)
    def _():
        st0 = (_NS - 1) * _STAGE
        pltpu.async_copy(q_hbm.at[pl.ds(st0, _LAST)],
                         q_sh.at[pl.ds(st0, _LAST)], stage_sem).wait()

    plsc.subcore_barrier()

    def idx_slice(t):
        return xr_hbm.at[pl.ds(base + t * _K, _K)]

    def out_slice(t):
        return out_hbm.at[pl.ds(base + t * _K, _K)]

    pltpu.async_copy(idx_slice(0), idx_a, isem_a)
    pltpu.async_copy(idx_slice(1), idx_b, isem_b)

    def half_step(t, last_t, idx_v, rows_v, isem, gsem, ssem):
        pltpu.make_async_copy(idx_slice(t), idx_v, isem).wait()

        @pl.when(t >= 2)
        def _():
            pltpu.make_async_copy(rows_v, out_slice(t - 2), ssem).wait()

        cps = [
            pltpu.async_copy(q_sh.at[idx_v.at[k]], rows_v.at[k], gsem)
            for k in range(_K)
        ]
        for cp in cps:
            cp.wait()

        @pl.when(t + 2 <= last_t)
        def _():
            pltpu.async_copy(idx_slice(t + 2), idx_v, isem)

        pltpu.async_copy(rows_v, out_slice(t), ssem)

    def dstep(m, carry):
        half_step(2 * m, _CHUNKS - 1, idx_a, rows_a, isem_a, gsem_a, ssem_a)
        half_step(2 * m + 1, _CHUNKS - 1, idx_b, rows_b, isem_b, gsem_b, ssem_b)
        return carry

    lax.fori_loop(0, _CHUNKS // 2, dstep, 0)

    pltpu.make_async_copy(rows_a, out_slice(_CHUNKS - 2), ssem_a).wait()
    pltpu.make_async_copy(rows_b, out_slice(_CHUNKS - 1), ssem_b).wait()


def _gather(xr, qp):
    mesh = plsc.VectorSubcoreMesh(core_axis_name="c", subcore_axis_name="s")
    kern = functools.partial(
        pl.kernel,
        mesh=mesh,
        out_type=jax.ShapeDtypeStruct((_NSTREAM, _IDX_PER_STREAM),
                                      jnp.float32),
        scratch_types=[
            pltpu.VMEM_SHARED((_V,), jnp.float32),
            pltpu.VMEM((_K, _IDX_PER_STREAM), jnp.int32),
            pltpu.VMEM((_K, _IDX_PER_STREAM), jnp.int32),
            pltpu.VMEM((_K, _IDX_PER_STREAM), jnp.float32),
            pltpu.VMEM((_K, _IDX_PER_STREAM), jnp.float32),
            pltpu.SemaphoreType.DMA,
            pltpu.SemaphoreType.DMA,
            pltpu.SemaphoreType.DMA,
            pltpu.SemaphoreType.DMA,
            pltpu.SemaphoreType.DMA,
            pltpu.SemaphoreType.DMA,
            pltpu.SemaphoreType.DMA,
        ],
        compiler_params=pltpu.CompilerParams(use_tc_tiling_on_sc=False),
    )(_gather_body)
    return kern(xr, qp)


def _head_body(g_ref, b_ref, out_ref, acc_ref):
    i = pl.program_id(0)
    part = jnp.sum(g_ref[...].reshape(25, 128, 128), axis=0)

    @pl.when(i == 0)
    def _():
        acc_ref[...] = part

    @pl.when(i > 0)
    def _():
        acc_ref[...] = acc_ref[...] + part

    @pl.when(i == pl.num_programs(0) - 1)
    def _():
        bb = b_ref[...]
        d = acc_ref[...] + (bb[0, 0] - bb[0, 1])
        r2 = lax.broadcasted_iota(jnp.int32, (_C, 128, 128), 0)
        z = jnp.where(r2 == 0, -d[None], d[None])
        sp = jnp.maximum(z, 0.0) + jnp.log1p(jnp.exp(-jnp.abs(z)))
        out_ref[...] = -sp


def _head(g, b2):
    rows_per_blk = 3200
    grid = _NSTREAM // rows_per_blk
    return pl.pallas_call(
        _head_body,
        grid=(grid,),
        in_specs=[
            pl.BlockSpec((rows_per_blk, _IDX_PER_STREAM), lambda i: (i, 0)),
            pl.BlockSpec((1, _C), lambda i: (0, 0)),
        ],
        out_specs=pl.BlockSpec((_C, 128, 128), lambda i: (0, 0, 0)),
        out_shape=jax.ShapeDtypeStruct((_C, 128, 128), jnp.float32),
        scratch_shapes=[pltpu.VMEM((128, 128), jnp.float32)],
    )(g, b2)


def kernel(x, emb_table, W, b):
    q = _project(emb_table.T, W.T)
    xr = x.T.reshape(_NSTREAM, _IDX_PER_STREAM)
    g = _gather(xr, q)
    out_t = _head(g, b.reshape(1, _C))
    return out_t.reshape(_C, _B).T


# speedup vs baseline: 7.1924x; 1.0165x over previous
import functools

import jax
import jax.numpy as jnp
from jax import lax
from jax.experimental import pallas as pl
from jax.experimental.pallas import tpu as pltpu
from jax.experimental.pallas import tpu_sc as plsc

_B = 16384
_H = 200
_V = 1000000
_D = 64
_C = 2

_NC = 2
_NS = 16
_NW = _NC * _NS

_IDX_PER_STREAM = 128
_NSTREAM = (_B * _H) // _IDX_PER_STREAM
_STREAMS_PER_TILE = _NSTREAM // _NW
_K = 25
_CHUNKS = _STREAMS_PER_TILE // _K

_STAGE = 62504
_LAST = _V - 15 * _STAGE


def _project_body(embt_ref, wt_ref, out_ref):
    e = embt_ref[...]
    wt = wt_ref[...]
    dwv = (wt[:, 0:1] - wt[:, 1:2]) * (1.0 / _H)
    p = lax.dot_general(dwv, e, (((0,), (0,)), ((), ())),
                        preferred_element_type=jnp.float32)
    out_ref[...] = p.reshape(p.shape[1])


def _project(embt, wt):
    blk = 51200
    grid = -(-_V // blk)
    return pl.pallas_call(
        _project_body,
        grid=(grid,),
        in_specs=[
            pl.BlockSpec((_D, blk), lambda i: (0, i)),
            pl.BlockSpec((_D, _C), lambda i: (0, 0)),
        ],
        out_specs=pl.BlockSpec((blk,), lambda i: (i,)),
        out_shape=jax.ShapeDtypeStruct((_V,), jnp.float32),
    )(embt, wt)


def _gather_body(xr_hbm, q_hbm, out_hbm,
                 q_sh, idx_a, idx_b, rows_a, rows_b,
                 stage_sem, isem_a, isem_b, gsem_a, gsem_b, ssem_a, ssem_b):
    sid = lax.axis_index("s")
    wid = sid * _NC + lax.axis_index("c")
    base = wid * _STREAMS_PER_TILE

    @pl.when(sid < _NS - 1)
    def _():
        st0 = sid * _STAGE
        pltpu.async_copy(q_hbm.at[pl.ds(st0, _STAGE)],
                         q_sh.at[pl.ds(st0, _STAGE)], stage_sem).wait()

    @pl.when(sid == _NS - 1)
    def _():
        st0 = (_NS - 1) * _STAGE
        pltpu.async_copy(q_hbm.at[pl.ds(st0, _LAST)],
                         q_sh.at[pl.ds(st0, _LAST)], stage_sem).wait()

    plsc.subcore_barrier()

    def idx_slice(t):
        return xr_hbm.at[pl.ds(base + t * _K, _K)]

    def out_slice(t):
        return out_hbm.at[pl.ds(base + t * _K, _K)]

    pltpu.async_copy(idx_slice(0), idx_a, isem_a)
    pltpu.async_copy(idx_slice(1), idx_b, isem_b)

    def half_step(t, last_t, idx_v, rows_v, isem, gsem, ssem):
        pltpu.make_async_copy(idx_slice(t), idx_v, isem).wait()

        @pl.when(t >= 2)
        def _():
            pltpu.make_async_copy(rows_v, out_slice(t - 2), ssem).wait()

        cps = [
            pltpu.async_copy(q_sh.at[idx_v.at[k]], rows_v.at[k], gsem)
            for k in range(_K)
        ]
        for cp in cps:
            cp.wait()

        @pl.when(t + 2 <= last_t)
        def _():
            pltpu.async_copy(idx_slice(t + 2), idx_v, isem)

        pltpu.async_copy(rows_v, out_slice(t), ssem)

    def dstep(m, carry):
        half_step(2 * m, _CHUNKS - 1, idx_a, rows_a, isem_a, gsem_a, ssem_a)
        half_step(2 * m + 1, _CHUNKS - 1, idx_b, rows_b, isem_b, gsem_b, ssem_b)
        return carry

    lax.fori_loop(0, _CHUNKS // 2, dstep, 0)

    pltpu.make_async_copy(rows_a, out_slice(_CHUNKS - 2), ssem_a).wait()
    pltpu.make_async_copy(rows_b, out_slice(_CHUNKS - 1), ssem_b).wait()


def _gather(xr, qp):
    mesh = plsc.VectorSubcoreMesh(core_axis_name="c", subcore_axis_name="s")
    kern = functools.partial(
        pl.kernel,
        mesh=mesh,
        out_type=jax.ShapeDtypeStruct((_NSTREAM, _IDX_PER_STREAM),
                                      jnp.float32),
        scratch_types=[
            pltpu.VMEM_SHARED((_V,), jnp.float32),
            pltpu.VMEM((_K, _IDX_PER_STREAM), jnp.int32),
            pltpu.VMEM((_K, _IDX_PER_STREAM), jnp.int32),
            pltpu.VMEM((_K, _IDX_PER_STREAM), jnp.float32),
            pltpu.VMEM((_K, _IDX_PER_STREAM), jnp.float32),
            pltpu.SemaphoreType.DMA,
            pltpu.SemaphoreType.DMA,
            pltpu.SemaphoreType.DMA,
            pltpu.SemaphoreType.DMA,
            pltpu.SemaphoreType.DMA,
            pltpu.SemaphoreType.DMA,
            pltpu.SemaphoreType.DMA,
        ],
        compiler_params=pltpu.CompilerParams(use_tc_tiling_on_sc=False),
    )(_gather_body)
    return kern(xr, qp)


def _head_body(g_ref, b_ref, out_ref, acc_ref):
    i = pl.program_id(0)
    part = jnp.sum(g_ref[...].reshape(25, 128, 128), axis=0)

    @pl.when(i == 0)
    def _():
        acc_ref[...] = part

    @pl.when(i > 0)
    def _():
        acc_ref[...] = acc_ref[...] + part

    @pl.when(i == pl.num_programs(0) - 1)
    def _():
        bb = b_ref[...]
        d = acc_ref[...] + (bb[0, 0] - bb[0, 1])
        r2 = lax.broadcasted_iota(jnp.int32, (_C, 128, 128), 0)
        z = jnp.where(r2 == 0, -d[None], d[None])
        sp = jnp.maximum(z, 0.0) + jnp.log1p(jnp.exp(-jnp.abs(z)))
        out_ref[...] = -sp


def _head(g, b2):
    rows_per_blk = 3200
    grid = _NSTREAM // rows_per_blk
    return pl.pallas_call(
        _head_body,
        grid=(grid,),
        in_specs=[
            pl.BlockSpec((rows_per_blk, _IDX_PER_STREAM), lambda i: (i, 0)),
            pl.BlockSpec((1, _C), lambda i: (0, 0)),
        ],
        out_specs=pl.BlockSpec((_C, 128, 128), lambda i: (0, 0, 0)),
        out_shape=jax.ShapeDtypeStruct((_C, 128, 128), jnp.float32),
        scratch_shapes=[pltpu.VMEM((128, 128), jnp.float32)],
    )(g, b2)


def kernel(x, emb_table, W, b):
    q = _project(emb_table.T, W.T)
    xr = x.T.reshape(_NSTREAM, _IDX_PER_STREAM)
    g = _gather(xr, q)
    out_t = _head(g, b.reshape(1, _C))
    return out_t.reshape(_C, _B).T
